# baseline (speedup 1.0000x reference)
import functools

import jax
import jax.numpy as jnp
from jax import lax
from jax.experimental import pallas as pl
from jax.experimental.pallas import tpu as pltpu
from jax.experimental.pallas import tpu_sc as plsc

_INFO = plsc.get_sparse_core_info()
_NC, _NS, _L = _INFO.num_cores, _INFO.num_subcores, _INFO.num_lanes
_NW = _NC * _NS


@functools.partial(jax.jit, static_argnums=(2, 3))
def _sc_gather(table, fwd_flat, n_rows, c):
    rows_per_w = n_rows // _NW
    n_chunks = 36
    n_buf = 12
    chunk = rows_per_w // n_chunks
    b_cols = 64

    mesh = plsc.VectorSubcoreMesh(core_axis_name="c", subcore_axis_name="s")

    @functools.partial(
        pl.kernel,
        out_type=jax.ShapeDtypeStruct((n_rows, c), jnp.float32),
        mesh=mesh,
        scratch_types=[
            pltpu.VMEM((rows_per_w,), jnp.int32),
            pltpu.VMEM((rows_per_w,), jnp.int32),
            pltpu.VMEM((n_buf, chunk, c), jnp.float32),
            pltpu.SemaphoreType.DMA,
            pltpu.SemaphoreType.DMA,
            pltpu.SemaphoreType.DMA,
            pltpu.SemaphoreType.DMA,
        ],
    )
    def body(table_hbm, fwd_hbm, out_hbm, fwd_v, idx_v, rows_v,
             gs0, gs1, os0, os1):
        wid = lax.axis_index("s") * _NC + lax.axis_index("c")
        base = wid * rows_per_w
        pltpu.sync_copy(fwd_hbm.at[pl.ds(base, rows_per_w)], fwd_v)
        lane = lax.broadcasted_iota(jnp.int32, (_L,), 0)
        for j in range(rows_per_w // _L):
            pos = base + j * _L
            col = lax.rem(pos + lane, b_cols)
            idx_v[pl.ds(j * _L, _L)] = fwd_v[pl.ds(j * _L, _L)] * b_cols + col

        gsem, osem = (gs0, gs1), (os0, os1)

        def gather(ci):
            b = ci % n_buf
            return pltpu.async_copy(
                table_hbm.at[idx_v.at[pl.ds(ci * chunk, chunk)]],
                rows_v.at[b], gsem[b % len(gsem)])

        def put(ci):
            b = ci % n_buf
            return pltpu.async_copy(
                rows_v.at[b], out_hbm.at[pl.ds(base + ci * chunk, chunk)],
                osem[b % len(osem)])

        gh = [gather(k) for k in range(n_buf)]
        oh = [None] * n_chunks
        for ci in range(n_chunks):
            gh[ci % n_buf].wait()
            oh[ci] = put(ci)
            nxt = ci + n_buf - 1
            if ci >= 1 and nxt < n_chunks:
                oh[nxt - n_buf].wait()
                gh[nxt % n_buf] = gather(nxt)
        for ci in range(n_chunks - n_buf, n_chunks):
            oh[ci].wait()

    return body(table, fwd_flat)


def kernel(patches, forward_indexes, backward_indexes):
    t, b, c = patches.shape
    remain_t = t // 4
    table = patches.reshape(t * b, c)
    fwd_flat = forward_indexes.astype(jnp.int32).reshape(-1)
    kept = _sc_gather(table, fwd_flat, remain_t * b, c).reshape(remain_t, b, c)
    return (kept, forward_indexes, backward_indexes)


# speedup vs baseline: 1.0170x; 1.0170x over previous
import functools

import jax
import jax.numpy as jnp
from jax import lax
from jax.experimental import pallas as pl
from jax.experimental.pallas import tpu as pltpu
from jax.experimental.pallas import tpu_sc as plsc

_INFO = plsc.get_sparse_core_info()
_NC, _NS, _L = _INFO.num_cores, _INFO.num_subcores, _INFO.num_lanes
_NW = _NC * _NS


@functools.partial(jax.jit, static_argnums=(2, 3))
def _sc_gather(table, fwd_flat, n_rows, c):
    rows_per_w = n_rows // _NW
    n_chunks = 18
    n_buf = 6
    chunk = rows_per_w // n_chunks
    b_cols = 64

    mesh = plsc.VectorSubcoreMesh(core_axis_name="c", subcore_axis_name="s")

    @functools.partial(
        pl.kernel,
        out_type=jax.ShapeDtypeStruct((n_rows, c), jnp.float32),
        mesh=mesh,
        scratch_types=[
            pltpu.VMEM((rows_per_w,), jnp.int32),
            pltpu.VMEM((rows_per_w,), jnp.int32),
            pltpu.VMEM((n_buf, chunk, c), jnp.float32),
            pltpu.SemaphoreType.DMA,
            pltpu.SemaphoreType.DMA,
            pltpu.SemaphoreType.DMA,
            pltpu.SemaphoreType.DMA,
        ],
    )
    def body(table_hbm, fwd_hbm, out_hbm, fwd_v, idx_v, rows_v,
             gs0, gs1, os0, os1):
        wid = lax.axis_index("s") * _NC + lax.axis_index("c")
        base = wid * rows_per_w
        pltpu.sync_copy(fwd_hbm.at[pl.ds(base, rows_per_w)], fwd_v)
        lane = lax.broadcasted_iota(jnp.int32, (_L,), 0)
        for j in range(rows_per_w // _L):
            pos = base + j * _L
            col = lax.rem(pos + lane, b_cols)
            idx_v[pl.ds(j * _L, _L)] = fwd_v[pl.ds(j * _L, _L)] * b_cols + col

        gsem, osem = (gs0, gs1), (os0, os1)

        def gather(ci):
            b = ci % n_buf
            return pltpu.async_copy(
                table_hbm.at[idx_v.at[pl.ds(ci * chunk, chunk)]],
                rows_v.at[b], gsem[b % len(gsem)])

        def put(ci):
            b = ci % n_buf
            return pltpu.async_copy(
                rows_v.at[b], out_hbm.at[pl.ds(base + ci * chunk, chunk)],
                osem[b % len(osem)])

        gh = [gather(k) for k in range(n_buf)]
        oh = [None] * n_chunks
        for ci in range(n_chunks):
            gh[ci % n_buf].wait()
            oh[ci] = put(ci)
            nxt = ci + n_buf - 1
            if ci >= 1 and nxt < n_chunks:
                oh[nxt - n_buf].wait()
                gh[nxt % n_buf] = gather(nxt)
        for ci in range(n_chunks - n_buf, n_chunks):
            oh[ci].wait()

    return body(table, fwd_flat)


def kernel(patches, forward_indexes, backward_indexes):
    t, b, c = patches.shape
    remain_t = t // 4
    table = patches.reshape(t * b, c)
    fwd_flat = forward_indexes.astype(jnp.int32).reshape(-1)
    kept = _sc_gather(table, fwd_flat, remain_t * b, c).reshape(remain_t, b, c)
    return (kept, forward_indexes, backward_indexes)
